# Initial kernel scaffold; baseline (speedup 1.0000x reference)
import jax
import jax.numpy as jnp
from jax.experimental import pallas as pl


def kernel(h, edge_index, edge_attr, batch, W0, Wih0, Whh0, bih0, bhh0, W1, Wih1, Whh1, bih1, bhh1, W2, Wih2, Whh2, bih2, bhh2, W3, Wih3, Whh3, bih3, bhh3, fc1_w, fc1_b, fc2_w, fc2_b):
    raise NotImplementedError("write your pallas kernel here")


# speedup vs baseline: 4.3704x; 4.3704x over previous
import functools

import jax
import jax.numpy as jnp
from jax import lax
from jax.experimental import pallas as pl
from jax.experimental.pallas import tpu as pltpu
from jax.experimental.pallas import tpu_sc as plsc

_N = 10000
_E = 320000
_D = 128
_G = 128
_C = 10
_H3 = 3 * _D

_NC, _NS = 2, 16
_NW = _NC * _NS
_EPW = _E // _NW
_CH = 80
_NCHUNK = _EPW // _CH
_NPAD = 10240
_RPT = _NPAD // _NS
_ZB = 64

_RB = 1000
_NBLK = _N // _RB


def _pre_body(x_ref, w_ref, whh_ref, bhh_ref, m_ref, gh_ref):
    x = x_ref[...]
    m_ref[...] = jnp.dot(x, w_ref[...], preferred_element_type=jnp.float32)
    gh_ref[...] = lax.dot_general(
        x, whh_ref[...], (((1,), (1,)), ((), ())),
        preferred_element_type=jnp.float32) + bhh_ref[...]


def _pre(x, W, Whh, bhh):
    return pl.pallas_call(
        _pre_body,
        grid=(_NBLK,),
        in_specs=[
            pl.BlockSpec((_RB, _D), lambda i: (i, 0)),
            pl.BlockSpec((_D, _D), lambda i: (0, 0)),
            pl.BlockSpec((_H3, _D), lambda i: (0, 0)),
            pl.BlockSpec((1, _H3), lambda i: (0, 0)),
        ],
        out_specs=[
            pl.BlockSpec((_RB, _D), lambda i: (i, 0)),
            pl.BlockSpec((_RB, _H3), lambda i: (i, 0)),
        ],
        out_shape=[
            jax.ShapeDtypeStruct((_N, _D), jnp.float32),
            jax.ShapeDtypeStruct((_N, _H3), jnp.float32),
        ],
    )(x, W, Whh, bhh.reshape(1, _H3))


def _sc_scatter(m, src, dst, zeros_zb):
    mesh = plsc.VectorSubcoreMesh(core_axis_name="c", subcore_axis_name="s")

    @functools.partial(
        pl.kernel,
        mesh=mesh,
        out_type=jax.ShapeDtypeStruct((_NC, _NPAD, _D), jnp.float32),
        scratch_types=[
            pltpu.VMEM((_CH,), jnp.int32),
            pltpu.VMEM((_CH,), jnp.int32),
            pltpu.VMEM((_CH, _D), jnp.float32),
            pltpu.VMEM((_ZB, _D), jnp.float32),
            pltpu.VMEM_SHARED((_NPAD, _D), jnp.float32),
            pltpu.SemaphoreType.DMA,
        ],
    )
    def k(m_hbm, src_hbm, dst_hbm, z_hbm, out_hbm,
          src_v, dst_v, rows_v, zbuf, agg_sh, sem):
        c = lax.axis_index("c")
        s = lax.axis_index("s")
        w = c * _NS + s
        pltpu.sync_copy(z_hbm, zbuf)
        row0 = s * _RPT
        for j in range(_RPT // _ZB):
            pltpu.sync_copy(zbuf, agg_sh.at[pl.ds(row0 + j * _ZB, _ZB), :])
        plsc.subcore_barrier()

        ebase = w * _EPW

        def body(i, carry):
            off = ebase + i * _CH
            pltpu.sync_copy(src_hbm.at[pl.ds(off, _CH)], src_v)
            pltpu.sync_copy(dst_hbm.at[pl.ds(off, _CH)], dst_v)
            pltpu.async_copy(m_hbm.at[src_v], rows_v, sem).wait()
            pltpu.sync_copy(rows_v, agg_sh.at[dst_v], add=True)
            return carry

        lax.fori_loop(0, _NCHUNK, body, 0)
        plsc.subcore_barrier()

        for j in range(_RPT // _CH):
            r0 = row0 + j * _CH
            pltpu.sync_copy(agg_sh.at[pl.ds(r0, _CH), :], rows_v)
            pltpu.sync_copy(rows_v, out_hbm.at[c, pl.ds(r0, _CH), :])

    return k(m, src, dst, zeros_zb)


def _post_body(agg0_ref, agg1_ref, gh_ref, x_ref, wih_ref, bih_ref, o_ref):
    agg = agg0_ref[0] + agg1_ref[0]
    gi = lax.dot_general(
        agg, wih_ref[...], (((1,), (1,)), ((), ())),
        preferred_element_type=jnp.float32) + bih_ref[...]
    gh = gh_ref[...]
    x = x_ref[...]
    r = jax.nn.sigmoid(gi[:, 0:_D] + gh[:, 0:_D])
    z = jax.nn.sigmoid(gi[:, _D:2 * _D] + gh[:, _D:2 * _D])
    n = jnp.tanh(gi[:, 2 * _D:] + r * gh[:, 2 * _D:])
    o_ref[...] = jnp.maximum((1.0 - z) * n + z * x, 0.0)


def _post(aggout, gh, x, Wih, bih):
    return pl.pallas_call(
        _post_body,
        grid=(_NBLK,),
        in_specs=[
            pl.BlockSpec((1, _RB, _D), lambda i: (0, i, 0)),
            pl.BlockSpec((1, _RB, _D), lambda i: (1, i, 0)),
            pl.BlockSpec((_RB, _H3), lambda i: (i, 0)),
            pl.BlockSpec((_RB, _D), lambda i: (i, 0)),
            pl.BlockSpec((_H3, _D), lambda i: (0, 0)),
            pl.BlockSpec((1, _H3), lambda i: (0, 0)),
        ],
        out_specs=pl.BlockSpec((_RB, _D), lambda i: (i, 0)),
        out_shape=jax.ShapeDtypeStruct((_N, _D), jnp.float32),
    )(aggout, aggout, gh, x, Wih, bih.reshape(1, _H3))


def _pool_body(x_ref, b_ref, fc1w_ref, fc1b_ref, fc2w_ref, fc2b_ref,
               o_ref, sums_ref, cnts_ref):
    i = pl.program_id(0)

    @pl.when(i == 0)
    def _():
        sums_ref[...] = jnp.zeros_like(sums_ref)
        cnts_ref[...] = jnp.zeros_like(cnts_ref)

    x = x_ref[...]
    b = b_ref[...]
    gids = lax.broadcasted_iota(jnp.int32, (_RB, _G), 1)
    onehot = (b == gids).astype(jnp.float32)
    sums_ref[...] += lax.dot_general(
        onehot, x, (((0,), (0,)), ((), ())),
        preferred_element_type=jnp.float32)
    cnts_ref[...] += lax.dot_general(
        onehot, jnp.ones((_RB, _G), jnp.float32), (((0,), (0,)), ((), ())),
        preferred_element_type=jnp.float32)

    @pl.when(i == _NBLK - 1)
    def _():
        hg = sums_ref[...] / jnp.maximum(cnts_ref[...], 1.0)
        hg = jnp.dot(hg, fc1w_ref[...],
                     preferred_element_type=jnp.float32) + fc1b_ref[...]
        hg = jnp.where(hg > 0, hg, jnp.exp(hg) - 1.0)
        hg = jnp.dot(hg, fc2w_ref[...],
                     preferred_element_type=jnp.float32) + fc2b_ref[...]
        mx = jnp.max(hg, axis=0, keepdims=True)
        lse = jnp.log(jnp.sum(jnp.exp(hg - mx), axis=0, keepdims=True))
        o_ref[...] = hg - mx - lse


def _pool(x, batch2d, fc1_w, fc1_b, fc2_w, fc2_b):
    return pl.pallas_call(
        _pool_body,
        grid=(_NBLK,),
        in_specs=[
            pl.BlockSpec((_RB, _D), lambda i: (i, 0)),
            pl.BlockSpec((_RB, 1), lambda i: (i, 0)),
            pl.BlockSpec((_D, _D), lambda i: (0, 0)),
            pl.BlockSpec((1, _D), lambda i: (0, 0)),
            pl.BlockSpec((_D, _C), lambda i: (0, 0)),
            pl.BlockSpec((1, _C), lambda i: (0, 0)),
        ],
        out_specs=pl.BlockSpec((_G, _C), lambda i: (0, 0)),
        out_shape=jax.ShapeDtypeStruct((_G, _C), jnp.float32),
        scratch_shapes=[
            pltpu.VMEM((_G, _G), jnp.float32),
            pltpu.VMEM((_G, _G), jnp.float32),
        ],
    )(x, batch2d, fc1_w, fc1_b.reshape(1, _D), fc2_w, fc2_b.reshape(1, _C))


def kernel(h, edge_index, edge_attr, batch,
           W0, Wih0, Whh0, bih0, bhh0,
           W1, Wih1, Whh1, bih1, bhh1,
           W2, Wih2, Whh2, bih2, bhh2,
           W3, Wih3, Whh3, bih3, bhh3,
           fc1_w, fc1_b, fc2_w, fc2_b):
    src = edge_index[0]
    dst = edge_index[1]
    zeros_zb = jnp.zeros((_ZB, _D), jnp.float32)
    params = [
        (W0, Wih0, Whh0, bih0, bhh0),
        (W1, Wih1, Whh1, bih1, bhh1),
        (W2, Wih2, Whh2, bih2, bhh2),
        (W3, Wih3, Whh3, bih3, bhh3),
    ]
    x = h
    for (W, Wih, Whh, bih, bhh) in params:
        m, gh = _pre(x, W, Whh, bhh)
        aggout = _sc_scatter(m, src, dst, zeros_zb)
        x = _post(aggout, gh, x, Wih, bih)
    return _pool(x, batch.reshape(_N, 1), fc1_w, fc1_b, fc2_w, fc2_b)
